# baseline (speedup 1.0000x reference)
import jax
import jax.numpy as jnp
from jax import lax
from jax.experimental import pallas as pl
from jax.experimental.pallas import tpu as pltpu, tpu_sc as plsc

VOCAB = 23236
VOCABP = 23296
DIM = 1728
BATCH = 1024
SEQ = 50

NC = 2
NS = 16
NW = NC * NS
NDT = DIM // 8
NUNIT = NDT * 4
KMAX = NUNIT // NW
TBLK = 512


def _unit_d0(wid, k):
    u = wid + NW * k
    dt = u // 4
    q = u % 4
    return dt, q, dt * 8 + q * 2


def _sc_body(ids_hbm, ttp_hbm, out_hbm, band0, band1, ids0, ids1, st0, st1,
             bsem0, bsem1, isem0, isem1, wsem0, wsem1):
    wid = lax.axis_index("s") * NC + lax.axis_index("c")

    def band_load(k, band, bsem):
        _, _, d0 = _unit_d0(wid, k)
        pltpu.async_copy(ttp_hbm.at[pl.ds(d0, 2)], band, bsem)

    def band_wait(k, band, bsem):
        _, _, d0 = _unit_d0(wid, k)
        pltpu.make_async_copy(ttp_hbm.at[pl.ds(d0, 2)], band, bsem).wait()

    def ids_load(s, idsb, isem):
        pltpu.async_copy(ids_hbm.at[s], idsb, isem)

    def ids_wait(s, idsb, isem):
        pltpu.make_async_copy(ids_hbm.at[s], idsb, isem).wait()

    def do_s(s, band, idsb, stage, wsem, dt, q):
        def g_body(g, carry):
            idx16 = idsb[pl.ds(g * 16, 16)]
            bt = g // 8
            off = (g % 8) * 16
            for ds in range(2):
                vals = plsc.load_gather(
                    band, [jnp.full((16,), ds, jnp.int32), idx16]
                )
                stage[bt, ds, pl.ds(off, 16)] = vals
            return carry

        lax.fori_loop(0, 64, g_body, 0, unroll=4)
        pltpu.async_copy(
            stage, out_hbm.at[s, dt, :, pl.ds(q * 2, 2), :], wsem
        )

    def write_wait(stage, wsem, dt, q):
        pltpu.make_async_copy(
            stage, out_hbm.at[0, dt, :, pl.ds(q * 2, 2), :], wsem
        ).wait()

    def do_unit(k, band, bsem):
        dt, q, _ = _unit_d0(wid, k)
        ids_load(0, ids0, isem0)
        ids_load(1, ids1, isem1)
        band_wait(k, band, bsem)
        ids_wait(0, ids0, isem0)
        do_s(0, band, ids0, st0, wsem0, dt, q)
        ids_load(2, ids0, isem0)
        ids_wait(1, ids1, isem1)
        do_s(1, band, ids1, st1, wsem1, dt, q)
        ids_load(3, ids1, isem1)

        def s_body(s2, carry):
            sa = 2 * s2
            write_wait(st0, wsem0, dt, q)
            ids_wait(sa, ids0, isem0)
            do_s(sa, band, ids0, st0, wsem0, dt, q)
            ids_load(jnp.minimum(sa + 2, SEQ - 1), ids0, isem0)
            sb = sa + 1
            write_wait(st1, wsem1, dt, q)
            ids_wait(sb, ids1, isem1)
            do_s(sb, band, ids1, st1, wsem1, dt, q)
            ids_load(jnp.minimum(sb + 2, SEQ - 1), ids1, isem1)
            return carry

        lax.fori_loop(1, SEQ // 2, s_body, 0)
        write_wait(st0, wsem0, dt, q)
        write_wait(st1, wsem1, dt, q)
        ids_wait(0, ids0, isem0)
        ids_wait(0, ids1, isem1)

    band_load(0, band0, bsem0)
    band_load(1, band1, bsem1)

    def k_body(k2, carry):
        do_unit(2 * k2, band0, bsem0)
        band_load(2 * k2 + 2, band0, bsem0)

        do_unit(2 * k2 + 1, band1, bsem1)

        @pl.when(k2 < KMAX // 2 - 1)
        def _():
            band_load(2 * k2 + 3, band1, bsem1)

        return carry

    lax.fori_loop(0, KMAX // 2, k_body, 0)
    do_unit(KMAX - 1, band0, bsem0)


def _sc_tgather(idsT, ttp):
    mesh = plsc.VectorSubcoreMesh(core_axis_name="c", subcore_axis_name="s")
    scratch = (
        [pltpu.VMEM((2, VOCABP), jnp.float32) for _ in range(2)]
        + [pltpu.VMEM((BATCH,), jnp.int32) for _ in range(2)]
        + [pltpu.VMEM((8, 2, 128), jnp.float32) for _ in range(2)]
        + [pltpu.SemaphoreType.DMA for _ in range(6)]
    )
    fn = pl.kernel(
        _sc_body,
        out_type=jax.ShapeDtypeStruct((SEQ, NDT, 8, 8, 128), jnp.float32),
        mesh=mesh,
        scratch_types=scratch,
        compiler_params=pltpu.CompilerParams(
            use_tc_tiling_on_sc=False, needs_layout_passes=False
        ),
    )
    return fn(idsT, ttp)


def _tc_copy_body(x_ref, o_ref):
    o_ref[...] = x_ref[...]


def _tc_compact(tT):
    grid = (VOCABP + TBLK - 1) // TBLK
    return pl.pallas_call(
        _tc_copy_body,
        grid=(grid,),
        in_specs=[pl.BlockSpec((DIM, TBLK), lambda i: (0, i))],
        out_specs=pl.BlockSpec((DIM, TBLK), lambda i: (0, i)),
        out_shape=jax.ShapeDtypeStruct((DIM, VOCABP), jnp.float32),
    )(tT)


@jax.jit
def _run(input_ids, table):
    ttp = _tc_compact(table.T)
    idsT = input_ids.T.astype(jnp.int32)
    o5 = _sc_tgather(idsT, ttp)
    out3 = jnp.transpose(o5, (0, 1, 3, 2, 4)).reshape(SEQ, DIM, BATCH)
    return jnp.transpose(out3, (2, 0, 1))


def kernel(input_ids, table):
    return _run(input_ids, table)
